# Initial kernel scaffold; baseline (speedup 1.0000x reference)
import jax
import jax.numpy as jnp
from jax.experimental import pallas as pl


def kernel(X, params):
    raise NotImplementedError("write your pallas kernel here")


# speedup vs baseline: 77.2269x; 77.2269x over previous
import jax
import jax.numpy as jnp
from jax import lax
from jax.experimental import pallas as pl
from jax.experimental.pallas import tpu as pltpu
from jax.experimental.pallas import tpu_sc as plsc

ARITY = 4
GROUPS = 512
OUT_DIM = 16
N_TOK = 2048
NVALS = 81
REF_IND = 40
NC, NS, L = 2, 16, 16
NW = NC * NS
GPW = GROUPS // NW
T_CHUNK = 128
N_CHUNKS = N_TOK // T_CHUNK
XCOLS = GPW * ARITY
XSLICE = 128
OCOLS = GPW * OUT_DIM
POW3 = (1, 3, 9, 27)


def _body(x_hbm, p_hbm, out_hbm, params_v, x_v, out_v):
    wid = lax.axis_index("s") * NC + lax.axis_index("c")
    g0 = wid * GPW

    pltpu.sync_copy(p_hbm.at[pl.ds(g0 * NVALS * OUT_DIM, GPW * NVALS * OUT_DIM)], params_v)

    xbase = (g0 * ARITY) // XSLICE * XSLICE
    xoff = g0 * ARITY - xbase

    iota = lax.iota(jnp.int32, L)
    iota4 = iota * ARITY + xoff
    oiota = iota * OUT_DIM
    g_off = iota * (NVALS * OUT_DIM)

    def token_body(t, carry):
        tv = jnp.full((L,), t, jnp.int32)
        a = [plsc.load_gather(x_v, [tv, iota4 + j]) for j in range(ARITY)]
        va = [jnp.abs(x) for x in a]
        tc = [
            jnp.where(a[j] >= 0.0, jnp.int32(POW3[j]), jnp.int32(-POW3[j]))
            for j in range(ARITY)
        ]

        def cswap(i, k):
            m = va[i] <= va[k]
            lo = jnp.minimum(va[i], va[k])
            hi = jnp.maximum(va[i], va[k])
            tlo = jnp.where(m, tc[i], tc[k])
            thi = jnp.where(m, tc[k], tc[i])
            va[i], va[k] = lo, hi
            tc[i], tc[k] = tlo, thi

        cswap(0, 1)
        cswap(2, 3)
        cswap(0, 2)
        cswap(1, 3)
        cswap(1, 2)

        c = [va[0], va[1] - va[0], va[2] - va[1], va[3] - va[2]]
        i3 = tc[3] + REF_IND
        i2 = i3 + tc[2]
        i1 = i2 + tc[1]
        i0 = i1 + tc[0]
        inds = [i0, i1, i2, i3]

        base = [g_off + inds[j] * OUT_DIM for j in range(ARITY)]
        for d in range(OUT_DIM):
            acc = c[0] * plsc.load_gather(params_v, [base[0] + d])
            for j in range(1, ARITY):
                acc = acc + c[j] * plsc.load_gather(params_v, [base[j] + d])
            plsc.store_scatter(out_v, [tv, oiota + d], acc)
        return carry

    def chunk_body(ci, carry):
        t0 = ci * T_CHUNK
        pltpu.sync_copy(
            x_hbm.at[pl.ds(t0, T_CHUNK), pl.ds(xbase, XSLICE)], x_v
        )
        lax.fori_loop(0, T_CHUNK, token_body, 0)
        pltpu.sync_copy(
            out_v, out_hbm.at[pl.ds(t0, T_CHUNK), pl.ds(g0 * OUT_DIM, OCOLS)]
        )
        return carry

    lax.fori_loop(0, N_CHUNKS, chunk_body, 0)


@jax.jit
def kernel(X, params):
    mesh = plsc.VectorSubcoreMesh(core_axis_name="c", subcore_axis_name="s")
    f = pl.kernel(
        _body,
        out_type=jax.ShapeDtypeStruct((N_TOK, GROUPS * OUT_DIM), jnp.float32),
        mesh=mesh,
        compiler_params=pltpu.CompilerParams(needs_layout_passes=False),
        scratch_types=[
            pltpu.VMEM((GPW * NVALS * OUT_DIM,), jnp.float32),
            pltpu.VMEM((T_CHUNK, XSLICE), jnp.float32),
            pltpu.VMEM((T_CHUNK, OCOLS), jnp.float32),
        ],
    )
    return f(X, params.reshape(-1))


# speedup vs baseline: 103.6877x; 1.3426x over previous
import jax
import jax.numpy as jnp
from jax import lax
from jax.experimental import pallas as pl
from jax.experimental.pallas import tpu as pltpu
from jax.experimental.pallas import tpu_sc as plsc

ARITY = 4
GROUPS = 512
OUT_DIM = 16
N_TOK = 2048
NVALS = 81
REF_IND = 40
NC, NS, L = 2, 16, 16
NW = NC * NS
GPW = GROUPS // NW
T_CHUNK = 128
N_CHUNKS = N_TOK // T_CHUNK
XCOLS = GPW * ARITY
XSLICE = 128
OCOLS = GPW * OUT_DIM
POW3 = (1, 3, 9, 27)


def _body(x_hbm, p_hbm, out_hbm, params_v, x_v, out_v):
    wid = lax.axis_index("s") * NC + lax.axis_index("c")
    g0 = wid * GPW

    pltpu.sync_copy(p_hbm.at[pl.ds(g0 * NVALS * OUT_DIM, GPW * NVALS * OUT_DIM)], params_v)

    xbase = (g0 * ARITY) // XSLICE * XSLICE
    xoff = g0 * ARITY - xbase

    iota = lax.iota(jnp.int32, L)
    iota4 = iota * ARITY + xoff
    oiota = iota * OUT_DIM
    g_off = iota * (NVALS * OUT_DIM)

    def token_body(t):
        tv = jnp.full((L,), t, jnp.int32)
        a = [plsc.load_gather(x_v, [tv, iota4 + j]) for j in range(ARITY)]
        va = [jnp.abs(x) for x in a]
        tc = [
            jnp.where(a[j] >= 0.0, jnp.int32(POW3[j]), jnp.int32(-POW3[j]))
            for j in range(ARITY)
        ]

        def cswap(i, k):
            m = va[i] <= va[k]
            lo = jnp.minimum(va[i], va[k])
            hi = jnp.maximum(va[i], va[k])
            tlo = jnp.where(m, tc[i], tc[k])
            thi = jnp.where(m, tc[k], tc[i])
            va[i], va[k] = lo, hi
            tc[i], tc[k] = tlo, thi

        cswap(0, 1)
        cswap(2, 3)
        cswap(0, 2)
        cswap(1, 3)
        cswap(1, 2)

        c = [va[0], va[1] - va[0], va[2] - va[1], va[3] - va[2]]
        i3 = tc[3] + REF_IND
        i2 = i3 + tc[2]
        i1 = i2 + tc[1]
        i0 = i1 + tc[0]
        inds = [i0, i1, i2, i3]

        base = [g_off + inds[j] * OUT_DIM for j in range(ARITY)]
        for d in range(OUT_DIM):
            r = [plsc.load_gather(params_v, [base[j] + d]) for j in range(ARITY)]
            acc = (c[0] * r[0] + c[1] * r[1]) + (c[2] * r[2] + c[3] * r[3])
            plsc.store_scatter(out_v, [tv, oiota + d], acc)

    def chunk_body(ci, carry):
        t0 = ci * T_CHUNK
        pltpu.sync_copy(
            x_hbm.at[pl.ds(t0, T_CHUNK), pl.ds(xbase, XSLICE)], x_v
        )
        plsc.parallel_loop(0, T_CHUNK, 1, unroll=4)(token_body)
        pltpu.sync_copy(
            out_v, out_hbm.at[pl.ds(t0, T_CHUNK), pl.ds(g0 * OUT_DIM, OCOLS)]
        )
        return carry

    lax.fori_loop(0, N_CHUNKS, chunk_body, 0)


@jax.jit
def kernel(X, params):
    mesh = plsc.VectorSubcoreMesh(core_axis_name="c", subcore_axis_name="s")
    f = pl.kernel(
        _body,
        out_type=jax.ShapeDtypeStruct((N_TOK, GROUPS * OUT_DIM), jnp.float32),
        mesh=mesh,
        compiler_params=pltpu.CompilerParams(needs_layout_passes=False),
        scratch_types=[
            pltpu.VMEM((GPW * NVALS * OUT_DIM,), jnp.float32),
            pltpu.VMEM((T_CHUNK, XSLICE), jnp.float32),
            pltpu.VMEM((T_CHUNK, OCOLS), jnp.float32),
        ],
    )
    return f(X, params.reshape(-1))
